# baseline (speedup 1.0000x reference)
import functools

import jax
import jax.numpy as jnp
from jax import lax
from jax.experimental import pallas as pl
from jax.experimental.pallas import tpu as pltpu
from jax.experimental.pallas import tpu_sc as plsc

_INFO = plsc.get_sparse_core_info()
_NC = _INFO.num_cores
_NW = _NC * _INFO.num_subcores

_CHUNK_ROWS = 64
_NBUF = 4


def kernel(img_batch, canvases):
    del canvases
    b, c, h, w = img_batch.shape
    rows = b * c * h
    flat = img_batch.reshape(rows, w)
    rpw = rows // _NW
    nchunks = rpw // _CHUNK_ROWS
    mesh = plsc.VectorSubcoreMesh(core_axis_name="c", subcore_axis_name="s")

    @functools.partial(
        pl.kernel,
        out_type=jax.ShapeDtypeStruct((rows, w), jnp.float32),
        mesh=mesh,
        scratch_types=[
            pltpu.VMEM_SHARED((_NBUF, 16 * _CHUNK_ROWS, w), jnp.float32),
            pltpu.SemaphoreType.DMA((_NBUF,)),
            pltpu.SemaphoreType.DMA((_NBUF,)),
        ],
    )
    def _sc_slot_copy(src_hbm, out_hbm, buf, in_sems, out_sems):
        sid = lax.axis_index("s")
        wid = sid * _NC + lax.axis_index("c")
        base = wid * rpw

        def in_copy(k):
            return pltpu.make_async_copy(
                src_hbm.at[pl.ds(base + k * _CHUNK_ROWS, _CHUNK_ROWS)],
                buf.at[k % _NBUF, pl.ds(sid * _CHUNK_ROWS, _CHUNK_ROWS)],
                in_sems.at[k % _NBUF],
            )

        def out_copy(k):
            return pltpu.make_async_copy(
                buf.at[k % _NBUF, pl.ds(sid * _CHUNK_ROWS, _CHUNK_ROWS)],
                out_hbm.at[pl.ds(base + k * _CHUNK_ROWS, _CHUNK_ROWS)],
                out_sems.at[k % _NBUF],
            )

        for k in range(_NBUF - 1):
            in_copy(k).start()
        for k in range(nchunks):
            in_copy(k).wait()
            out_copy(k).start()
            j = k + _NBUF - 1
            if j < nchunks:
                if k >= 1:
                    out_copy(k - 1).wait()
                in_copy(j).start()
        for k in range(max(0, nchunks - _NBUF), nchunks):
            out_copy(k).wait()

    return _sc_slot_copy(flat).reshape(b, c, h, w)


# speedup vs baseline: 1.0162x; 1.0162x over previous
import functools

import jax
import jax.numpy as jnp
from jax import lax
from jax.experimental import pallas as pl
from jax.experimental.pallas import tpu as pltpu
from jax.experimental.pallas import tpu_sc as plsc

_INFO = plsc.get_sparse_core_info()
_NC = _INFO.num_cores
_NS = _INFO.num_subcores
_NW = _NC * _NS

_CHUNK_ROWS = 64


def kernel(img_batch, canvases):
    del canvases
    b, c, h, w = img_batch.shape
    rows = b * c * h
    flat = img_batch.reshape(rows, w)
    rpw = rows // _NW
    nchunks = rpw // _CHUNK_ROWS
    nloc = nchunks // 2
    mesh = plsc.VectorSubcoreMesh(core_axis_name="c", subcore_axis_name="s")

    @functools.partial(
        pl.kernel,
        out_type=jax.ShapeDtypeStruct((rows, w), jnp.float32),
        mesh=mesh,
        scratch_types=[
            pltpu.VMEM((2, _CHUNK_ROWS, w), jnp.float32),
            pltpu.VMEM_SHARED((2, _NS * _CHUNK_ROWS, w), jnp.float32),
            pltpu.SemaphoreType.DMA((2,)),
            pltpu.SemaphoreType.DMA((2,)),
            pltpu.SemaphoreType.DMA((2,)),
            pltpu.SemaphoreType.DMA((2,)),
        ],
    )
    def _sc_slot_copy(src_hbm, out_hbm, buf_a, buf_b, in_a, out_a, in_b, out_b):
        sid = lax.axis_index("s")
        wid = sid * _NC + lax.axis_index("c")
        base = wid * rpw

        def hbm_slice(k):
            return src_hbm.at[pl.ds(base + k * _CHUNK_ROWS, _CHUNK_ROWS)]

        def hbm_out_slice(k):
            return out_hbm.at[pl.ds(base + k * _CHUNK_ROWS, _CHUNK_ROWS)]

        def stage(ring, slot):
            if ring == 0:
                return buf_a.at[slot]
            return buf_b.at[slot, pl.ds(sid * _CHUNK_ROWS, _CHUNK_ROWS)]

        def in_copy(ring, l):
            k = ring + 2 * l
            sem = in_a if ring == 0 else in_b
            return pltpu.make_async_copy(hbm_slice(k), stage(ring, l % 2), sem.at[l % 2])

        def out_copy(ring, l):
            k = ring + 2 * l
            sem = out_a if ring == 0 else out_b
            return pltpu.make_async_copy(stage(ring, l % 2), hbm_out_slice(k), sem.at[l % 2])

        in_copy(0, 0).start()
        in_copy(1, 0).start()
        for l in range(nloc):
            for ring in (0, 1):
                in_copy(ring, l).wait()
                if l + 1 < nloc:
                    if l >= 1:
                        out_copy(ring, l - 1).wait()
                    in_copy(ring, l + 1).start()
                out_copy(ring, l).start()
        for ring in (0, 1):
            out_copy(ring, nloc - 2).wait()
            out_copy(ring, nloc - 1).wait()

    return _sc_slot_copy(flat).reshape(b, c, h, w)
